# baseline (speedup 1.0000x reference)
import functools

import jax
import jax.numpy as jnp
from jax.experimental import pallas as pl
from jax.experimental.pallas import tpu as pltpu

_Q = 1024
_M = 65536
_BLK = 4096
_STEPS = _M // _BLK


def _tree_min_rows(x):
    r = x.shape[0]
    while r > 8:
        h = r // 2
        x = jnp.minimum(x[:h], x[h:])
        r = h
    return jnp.min(x, axis=0, keepdims=True)


def _knn_body(appt_ref, poset_ref, a2a_ref, a2p_ref, ma_ref, mp_ref,
              oa_ref, op_ref, acc_a, acc_p):
    j = pl.program_id(0)

    @pl.when(j == 0)
    def _init():
        acc_a[...] = jnp.full((1, _Q), jnp.inf, jnp.float32)
        acc_p[...] = jnp.full((1, _Q), jnp.inf, jnp.float32)

    ma = ma_ref[...].astype(jnp.bfloat16)
    maug = jnp.concatenate([ma, ma * ma], axis=1)
    ta = jnp.dot(maug, appt_ref[...],
                 preferred_element_type=jnp.float32)
    mina = _tree_min_rows(ta.astype(jnp.bfloat16)).astype(jnp.float32)
    acc_a[...] = jnp.minimum(acc_a[...], mina)

    mp = mp_ref[...].astype(jnp.bfloat16)
    paug = jnp.concatenate([mp, mp * mp], axis=1)
    tp = jnp.dot(paug, poset_ref[...],
                 preferred_element_type=jnp.float32)
    minp = _tree_min_rows(tp.astype(jnp.bfloat16)).astype(jnp.float32)
    acc_p[...] = jnp.minimum(acc_p[...], minp)

    @pl.when(j == _STEPS - 1)
    def _fin():
        oa_ref[...] = jnp.sqrt(jnp.maximum(a2a_ref[...] + acc_a[...], 0.0))
        op_ref[...] = jnp.sqrt(jnp.maximum(a2p_ref[...] + acc_p[...], 0.0))


@functools.partial(jax.jit, static_argnames=())
def kernel(app_features, pose_features, mem_app, mem_pose,
           norm_app_min, norm_app_max, norm_pose_min, norm_pose_max):
    app_t = jnp.concatenate(
        [(app_features * -2.0).T, jnp.ones_like(app_features).T],
        axis=0).astype(jnp.bfloat16)
    pose_t = jnp.concatenate(
        [(pose_features * -2.0).T, jnp.ones_like(pose_features).T],
        axis=0).astype(jnp.bfloat16)
    a2a = jnp.sum(app_features * app_features, axis=1)[None, :]
    a2p = jnp.sum(pose_features * pose_features, axis=1)[None, :]

    dist_a, dist_p = pl.pallas_call(
        _knn_body,
        grid=(_STEPS,),
        in_specs=[
            pl.BlockSpec((512, _Q), lambda j: (0, 0)),
            pl.BlockSpec((128, _Q), lambda j: (0, 0)),
            pl.BlockSpec((1, _Q), lambda j: (0, 0)),
            pl.BlockSpec((1, _Q), lambda j: (0, 0)),
            pl.BlockSpec((_BLK, 256), lambda j: (j, 0)),
            pl.BlockSpec((_BLK, 64), lambda j: (j, 0)),
        ],
        out_specs=[
            pl.BlockSpec((1, _Q), lambda j: (0, 0)),
            pl.BlockSpec((1, _Q), lambda j: (0, 0)),
        ],
        out_shape=[
            jax.ShapeDtypeStruct((1, _Q), jnp.float32),
            jax.ShapeDtypeStruct((1, _Q), jnp.float32),
        ],
        scratch_shapes=[
            pltpu.VMEM((1, _Q), jnp.float32),
            pltpu.VMEM((1, _Q), jnp.float32),
        ],
        compiler_params=pltpu.CompilerParams(
            dimension_semantics=("arbitrary",),
        ),
    )(app_t, pose_t, a2a, a2p, mem_app, mem_pose)

    score_a = (dist_a[0] - norm_app_min[0]) / (norm_app_max[0] - norm_app_min[0])
    score_p = (dist_p[0] - norm_pose_min[0]) / (norm_pose_max[0] - norm_pose_min[0])
    return score_a + score_p


# speedup vs baseline: 1.3451x; 1.3451x over previous
import functools

import jax
import jax.numpy as jnp
from jax.experimental import pallas as pl
from jax.experimental.pallas import tpu as pltpu

_Q = 1024
_M = 65536
_BLK = 4096
_STEPS = _M // _BLK


def _tree_min_rows(x):
    r = x.shape[0]
    while r > 8:
        h = r // 2
        x = jnp.minimum(x[:h], x[h:])
        r = h
    return jnp.min(x, axis=0, keepdims=True)


def _knn_body(appt_ref, poset_ref, a2a_ref, a2p_ref, ma_ref, mp_ref,
              oa_ref, op_ref, acc_a, acc_p):
    j = pl.program_id(0)

    @pl.when(j == 0)
    def _init():
        acc_a[...] = jnp.full((1, _Q), jnp.inf, jnp.float32)
        acc_p[...] = jnp.full((1, _Q), jnp.inf, jnp.float32)

    ma = ma_ref[...]
    b2a = jnp.sum(ma * ma, axis=1, keepdims=True)
    ga = jnp.dot(ma, appt_ref[...],
                 preferred_element_type=jnp.float32)
    ta = ga.astype(jnp.bfloat16) + b2a.astype(jnp.bfloat16)
    mina = _tree_min_rows(ta).astype(jnp.float32)
    acc_a[...] = jnp.minimum(acc_a[...], mina)

    mp = mp_ref[...]
    b2p = jnp.sum(mp * mp, axis=1, keepdims=True)
    gp = jnp.dot(mp, poset_ref[...],
                 preferred_element_type=jnp.float32)
    tp = gp.astype(jnp.bfloat16) + b2p.astype(jnp.bfloat16)
    minp = _tree_min_rows(tp).astype(jnp.float32)
    acc_p[...] = jnp.minimum(acc_p[...], minp)

    @pl.when(j == _STEPS - 1)
    def _fin():
        oa_ref[...] = jnp.sqrt(jnp.maximum(a2a_ref[...] + acc_a[...], 0.0))
        op_ref[...] = jnp.sqrt(jnp.maximum(a2p_ref[...] + acc_p[...], 0.0))


@functools.partial(jax.jit, static_argnames=())
def kernel(app_features, pose_features, mem_app, mem_pose,
           norm_app_min, norm_app_max, norm_pose_min, norm_pose_max):
    app_t = (app_features * -2.0).T
    pose_t = (pose_features * -2.0).T
    a2a = jnp.sum(app_features * app_features, axis=1)[None, :]
    a2p = jnp.sum(pose_features * pose_features, axis=1)[None, :]

    dist_a, dist_p = pl.pallas_call(
        _knn_body,
        grid=(_STEPS,),
        in_specs=[
            pl.BlockSpec((256, _Q), lambda j: (0, 0)),
            pl.BlockSpec((64, _Q), lambda j: (0, 0)),
            pl.BlockSpec((1, _Q), lambda j: (0, 0)),
            pl.BlockSpec((1, _Q), lambda j: (0, 0)),
            pl.BlockSpec((_BLK, 256), lambda j: (j, 0)),
            pl.BlockSpec((_BLK, 64), lambda j: (j, 0)),
        ],
        out_specs=[
            pl.BlockSpec((1, _Q), lambda j: (0, 0)),
            pl.BlockSpec((1, _Q), lambda j: (0, 0)),
        ],
        out_shape=[
            jax.ShapeDtypeStruct((1, _Q), jnp.float32),
            jax.ShapeDtypeStruct((1, _Q), jnp.float32),
        ],
        scratch_shapes=[
            pltpu.VMEM((1, _Q), jnp.float32),
            pltpu.VMEM((1, _Q), jnp.float32),
        ],
        compiler_params=pltpu.CompilerParams(
            dimension_semantics=("arbitrary",),
        ),
    )(app_t, pose_t, a2a, a2p, mem_app, mem_pose)

    score_a = (dist_a[0] - norm_app_min[0]) / (norm_app_max[0] - norm_app_min[0])
    score_p = (dist_p[0] - norm_pose_min[0]) / (norm_pose_max[0] - norm_pose_min[0])
    return score_a + score_p


# speedup vs baseline: 1.3676x; 1.0167x over previous
import functools

import jax
import jax.numpy as jnp
from jax.experimental import pallas as pl
from jax.experimental.pallas import tpu as pltpu

_Q = 1024
_M = 65536
_BLK = 4096
_STEPS = _M // _BLK


def _tree_min_rows(x):
    r = x.shape[0]
    while r > 8:
        h = r // 2
        x = jnp.minimum(x[:h], x[h:])
        r = h
    return jnp.min(x, axis=0, keepdims=True)


def _sum_sq_rows(x):
    x = x * x
    r = x.shape[0]
    while r > 1:
        h = r // 2
        x = x[:h] + x[h:]
        r = h
    return x


def _knn_body(app_ref, pose_ref, ma_ref, mp_ref, oa_ref, op_ref,
              appt_s, poset_s, acc_a, acc_p):
    j = pl.program_id(0)

    @pl.when(j == 0)
    def _init():
        appt_s[...] = app_ref[...].T
        poset_s[...] = pose_ref[...].T
        acc_a[...] = jnp.full((1, _Q), jnp.inf, jnp.float32)
        acc_p[...] = jnp.full((1, _Q), jnp.inf, jnp.float32)

    ma = ma_ref[...]
    b2a = 0.5 * jnp.sum(ma * ma, axis=1, keepdims=True)
    ga = jnp.dot(ma, appt_s[...],
                 preferred_element_type=jnp.float32)
    ta = b2a.astype(jnp.bfloat16) - ga.astype(jnp.bfloat16)
    mina = _tree_min_rows(ta).astype(jnp.float32)
    acc_a[...] = jnp.minimum(acc_a[...], mina)

    mp = mp_ref[...]
    b2p = 0.5 * jnp.sum(mp * mp, axis=1, keepdims=True)
    gp = jnp.dot(mp, poset_s[...],
                 preferred_element_type=jnp.float32)
    tp = b2p.astype(jnp.bfloat16) - gp.astype(jnp.bfloat16)
    minp = _tree_min_rows(tp).astype(jnp.float32)
    acc_p[...] = jnp.minimum(acc_p[...], minp)

    @pl.when(j == _STEPS - 1)
    def _fin():
        a2a = _sum_sq_rows(appt_s[...])
        a2p = _sum_sq_rows(poset_s[...])
        oa_ref[...] = jnp.sqrt(jnp.maximum(a2a + 2.0 * acc_a[...], 0.0))
        op_ref[...] = jnp.sqrt(jnp.maximum(a2p + 2.0 * acc_p[...], 0.0))


@functools.partial(jax.jit, static_argnames=())
def kernel(app_features, pose_features, mem_app, mem_pose,
           norm_app_min, norm_app_max, norm_pose_min, norm_pose_max):
    dist_a, dist_p = pl.pallas_call(
        _knn_body,
        grid=(_STEPS,),
        in_specs=[
            pl.BlockSpec((_Q, 256), lambda j: (0, 0)),
            pl.BlockSpec((_Q, 64), lambda j: (0, 0)),
            pl.BlockSpec((_BLK, 256), lambda j: (j, 0)),
            pl.BlockSpec((_BLK, 64), lambda j: (j, 0)),
        ],
        out_specs=[
            pl.BlockSpec((1, _Q), lambda j: (0, 0)),
            pl.BlockSpec((1, _Q), lambda j: (0, 0)),
        ],
        out_shape=[
            jax.ShapeDtypeStruct((1, _Q), jnp.float32),
            jax.ShapeDtypeStruct((1, _Q), jnp.float32),
        ],
        scratch_shapes=[
            pltpu.VMEM((256, _Q), jnp.float32),
            pltpu.VMEM((64, _Q), jnp.float32),
            pltpu.VMEM((1, _Q), jnp.float32),
            pltpu.VMEM((1, _Q), jnp.float32),
        ],
        compiler_params=pltpu.CompilerParams(
            dimension_semantics=("arbitrary",),
        ),
    )(app_features, pose_features, mem_app, mem_pose)

    score_a = (dist_a[0] - norm_app_min[0]) / (norm_app_max[0] - norm_app_min[0])
    score_p = (dist_p[0] - norm_pose_min[0]) / (norm_pose_max[0] - norm_pose_min[0])
    return score_a + score_p
